# baseline (speedup 1.0000x reference)
import functools

import jax
import jax.numpy as jnp
from jax import lax
from jax.experimental import pallas as pl
from jax.experimental.pallas import tpu as pltpu
from jax.experimental.pallas import tpu_sc as plsc

_TOPK = 64
_R = 1024
_N = 8192
_NW = 32
_ROWS_PW = _R // _NW
_NV = _N // 16
_CAP = _NV


def _f32_key(x):
    u = lax.bitcast_convert_type(x, jnp.uint32)
    return jnp.where((u >> 31) == 1, ~u, u | jnp.uint32(0x80000000))


def _key_to_f32(k):
    u = jnp.where(k >= jnp.uint32(0x80000000), k ^ jnp.uint32(0x80000000), ~k)
    return lax.bitcast_convert_type(u, jnp.float32)


def _row_topk(rowbuf, zbuf, idxbuf, keybuf):
    lane = lax.broadcasted_iota(jnp.int32, (16,), 0)
    lanebase = lane * _CAP
    one = jnp.ones((16,), jnp.int32)
    zero = jnp.zeros((16,), jnp.int32)

    ninf = jnp.full((16,), -jnp.inf, jnp.float32)

    def pass_a(i, accs):
        a0, a1, a2, a3 = accs
        j = i * 4
        a0 = jnp.maximum(a0, rowbuf[pl.ds(j * 16, 16)])
        a1 = jnp.maximum(a1, rowbuf[pl.ds((j + 1) * 16, 16)])
        a2 = jnp.maximum(a2, rowbuf[pl.ds((j + 2) * 16, 16)])
        a3 = jnp.maximum(a3, rowbuf[pl.ds((j + 3) * 16, 16)])
        return a0, a1, a2, a3

    a0, a1, a2, a3 = lax.fori_loop(
        0, _NV // 4, pass_a, (ninf, ninf, ninf, ninf), unroll=2
    )
    vmax = jnp.maximum(jnp.maximum(a0, a1), jnp.maximum(a2, a3))
    vmin = jnp.minimum(jnp.minimum(a0, a1), jnp.minimum(a2, a3))
    lb = jnp.min(vmin)
    lo0 = jnp.min(_f32_key(vmin))
    hi0 = jnp.max(_f32_key(vmax)) + jnp.uint32(1)

    def pass_b(i, percnt):
        for t in range(4):
            j = i * 4 + t
            x = rowbuf[pl.ds(j * 16, 16)]
            m = x >= lb
            plsc.store_scatter(idxbuf, [lanebase + percnt], lane + j * 16, mask=m)
            percnt = percnt + jnp.where(m, one, zero)
        return percnt

    percnt = lax.fori_loop(0, _NV // 4, pass_b, zero)
    maxc = jnp.max(percnt)

    def gather_keys(s, _):
        idxv = plsc.load_gather(idxbuf, [lanebase + s])
        xg = plsc.load_gather(rowbuf, [idxv])
        kv = jnp.where(percnt > s, _f32_key(xg), jnp.uint32(0))
        keybuf[pl.ds(s * 16, 16)] = kv
        return 0

    lax.fori_loop(0, maxc, gather_keys, 0)

    def bisect_cond(carry):
        lo, hi = carry
        return hi - lo > jnp.uint32(1)

    def bisect_body(carry):
        lo, hi = carry
        mid = lo + ((hi - lo) >> 1)

        def count_vec(s, acc):
            kv = keybuf[pl.ds(s * 16, 16)]
            return acc + (kv >= mid).astype(jnp.int32)

        acc = lax.fori_loop(0, maxc, count_vec, zero)
        c = jnp.sum(acc)
        return jnp.where(c >= _TOPK, mid, lo), jnp.where(c >= _TOPK, hi, mid)

    thresh, _ = lax.while_loop(bisect_cond, bisect_body, (lo0, hi0))

    def scatter_out(s, _):
        kv = keybuf[pl.ds(s * 16, 16)]
        keep = kv >= thresh
        idxv = plsc.load_gather(idxbuf, [lanebase + s])
        plsc.store_scatter(zbuf, [idxv], _key_to_f32(kv), mask=keep)
        return 0

    lax.fori_loop(0, maxc, scatter_out, 0)
    return maxc, thresh


def _restore_zeros(zbuf, idxbuf, keybuf, maxc, thresh):
    lane = lax.broadcasted_iota(jnp.int32, (16,), 0)
    lanebase = lane * _CAP
    zeros_f = jnp.zeros((16,), jnp.float32)

    def unscatter(s, _):
        kv = keybuf[pl.ds(s * 16, 16)]
        keep = kv >= thresh
        idxv = plsc.load_gather(idxbuf, [lanebase + s])
        plsc.store_scatter(zbuf, [idxv], zeros_f, mask=keep)
        return 0

    lax.fori_loop(0, maxc, unscatter, 0)


def _sc_topk_body(attn_hbm, out_hbm, rowbuf, zbuf, idxbuf, keybuf):
    wid = lax.axis_index("s") * 2 + lax.axis_index("c")
    base = wid * _ROWS_PW
    zeros_f = jnp.zeros((16,), jnp.float32)

    def zero_bufs(j, _):
        zbuf[pl.ds(j * 16, 16)] = zeros_f
        idxbuf[pl.ds(j * 16, 16)] = jnp.zeros((16,), jnp.int32)
        return 0

    lax.fori_loop(0, _NV, zero_bufs, 0)

    def per_row(r, _):
        row = base + r
        pltpu.sync_copy(attn_hbm.at[row], rowbuf)
        maxc, thresh = _row_topk(rowbuf, zbuf, idxbuf, keybuf)
        pltpu.sync_copy(zbuf, out_hbm.at[row])
        _restore_zeros(zbuf, idxbuf, keybuf, maxc, thresh)
        return 0

    lax.fori_loop(0, _ROWS_PW, per_row, 0)


@functools.partial(jax.jit, static_argnums=())
def _sc_topk(flat):
    mesh = plsc.VectorSubcoreMesh(core_axis_name="c", subcore_axis_name="s")
    k = functools.partial(
        pl.kernel,
        mesh=mesh,
        out_type=jax.ShapeDtypeStruct((_R, _N), jnp.float32),
        scratch_types=[
            pltpu.VMEM((_N,), jnp.float32),
            pltpu.VMEM((_N,), jnp.float32),
            pltpu.VMEM((16 * _CAP,), jnp.int32),
            pltpu.VMEM((_N,), jnp.uint32),
        ],
        compiler_params=pltpu.CompilerParams(needs_layout_passes=False),
    )(_sc_topk_body)
    return k(flat)


def kernel(attn):
    mb, num_q, num_k = attn.shape
    flat = attn.reshape(mb * num_q, num_k)
    out = _sc_topk(flat)
    return out.reshape(mb, num_q, num_k)
